# baseline (speedup 1.0000x reference)
import jax
import jax.numpy as jnp
from jax import lax
from jax.experimental import pallas as pl
from jax.experimental.pallas import tpu as pltpu
from jax.experimental.pallas import tpu_sc as plsc

M = 1000000
D = 59
DP = 64
B = 262144

NC = 2
NS = 16
L = 16

R = 16384
RSH = 14
NFULL = M // R
REM = M - NFULL * R
NB = 62
SPLIT = 31

SLICE = B // NS
NV = SLICE // L
C = 128
TRASH = R
ALIGN_SLACK = SPLIT * 8
PV = SLICE + ALIGN_SLACK + C + L
TRASHB = PV - L

ROWS_PER_TILE = R // NS
NH = NB * L


def _body(mem, val, idx, out, idx_v, pos_v, hist_v, cnt_v, lidx, rows, acc, gsem):
    c = lax.axis_index("c")
    s = lax.axis_index("s")
    tile_base = s * SLICE
    blk_lo = SPLIT * c

    pltpu.sync_copy(idx.at[pl.ds(tile_base, SLICE)], idx_v)

    def _prefill(i, _):
        pos_v[pl.ds(i * L, L)] = jnp.zeros((L,), jnp.int32) + tile_base
        return 0

    lax.fori_loop(0, PV // L, _prefill, 0)

    def _zero(i, _):
        hist_v[pl.ds(i * L, L)] = jnp.zeros((L,), jnp.int32)
        return 0

    lax.fori_loop(0, NH // L, _zero, 0)

    def _hist(i, _):
        lanes = lax.iota(jnp.int32, L)
        iv = idx_v[pl.ds(i * L, L)]
        blk = lax.shift_right_logical(iv, RSH)
        cidx = blk * L + lanes
        plsc.addupdate_scatter(hist_v, [cidx], jnp.zeros((L,), jnp.int32) + 1)
        return 0

    lax.fori_loop(0, NV, _hist, 0)

    def _bases(j, base):
        lanes = lax.iota(jnp.int32, L)
        hv = hist_v[pl.ds(j * L, L)]
        o = j - blk_lo
        own = (o >= 0) & (o < SPLIT)
        csum = plsc.cumsum(hv)
        excl = csum - hv
        cur = jnp.where(own, base + excl, TRASHB + lanes)
        cnt_v[pl.ds(j * L, L)] = cur
        cnt8 = (jnp.sum(hv) + 7) & jnp.int32(-8)
        return base + jnp.where(own, cnt8, 0)

    lax.fori_loop(0, NB, _bases, jnp.int32(0))

    def _rank(i, _):
        lanes = lax.iota(jnp.int32, L)
        iv = idx_v[pl.ds(i * L, L)]
        blk = lax.shift_right_logical(iv, RSH)
        o = blk - blk_lo
        own = ((o >= 0) & (o < SPLIT)).astype(jnp.int32)
        cidx = blk * L + lanes
        cur = plsc.load_gather(cnt_v, [cidx])
        posvec = (tile_base + i * L) + lanes
        plsc.store_scatter(pos_v, [cur], posvec)
        plsc.addupdate_scatter(cnt_v, [cidx], own)
        return 0

    lax.fori_loop(0, NV, _rank, 0)

    def _block(k, base):
        blk = blk_lo + k
        lo = blk * R
        isp = blk >= NFULL

        @pl.when(jnp.logical_not(isp))
        def _():
            pltpu.sync_copy(
                mem.at[pl.ds(lo + s * ROWS_PER_TILE, ROWS_PER_TILE)],
                acc.at[pl.ds(s * ROWS_PER_TILE, ROWS_PER_TILE)],
            )

        @pl.when(isp & (s == 0))
        def _():
            pltpu.sync_copy(mem.at[pl.ds(lo, REM)], acc.at[pl.ds(0, REM)])

        plsc.subcore_barrier()

        cnt = jnp.sum(hist_v[pl.ds(blk * L, L)])
        end = base + cnt

        def _chunk(j, _):
            c0 = pl.multiple_of(base + j * C, 8)
            for kk in range(C // L):
                lanes = lax.iota(jnp.int32, L)
                slot = c0 + kk * L + lanes
                pv = pos_v[pl.ds(c0 + kk * L, L)]
                ival = plsc.load_gather(idx_v, [pv - tile_base])
                li = ival - lo
                li = jnp.where(slot < end, li, TRASH)
                lidx[0, pl.ds(kk * L, L)] = li
            pltpu.async_copy(val.at[pos_v.at[pl.ds(c0, C)]], rows, gsem).wait()
            pltpu.sync_copy(rows, acc.at[lidx.at[0]], add=True)
            return 0

        nch = lax.div(cnt + (C - 1), jnp.int32(C))
        lax.fori_loop(0, nch, _chunk, 0)

        plsc.subcore_barrier()

        @pl.when(jnp.logical_not(isp))
        def _():
            pltpu.sync_copy(
                acc.at[pl.ds(s * ROWS_PER_TILE, ROWS_PER_TILE)],
                out.at[pl.ds(lo + s * ROWS_PER_TILE, ROWS_PER_TILE)],
            )

        @pl.when(isp & (s == 0))
        def _():
            pltpu.sync_copy(acc.at[pl.ds(0, REM)], out.at[pl.ds(lo, REM)])

        return base + ((cnt + 7) & jnp.int32(-8))

    lax.fori_loop(0, SPLIT, _block, jnp.int32(0))


@jax.jit
def _scatter_add(mem, val, idx):
    mem64 = jnp.zeros((M, DP), jnp.float32).at[:, :D].set(mem)
    val64 = jnp.zeros((B, DP), jnp.float32).at[:, :D].set(val)
    mesh = plsc.VectorSubcoreMesh(core_axis_name="c", subcore_axis_name="s")
    out64 = pl.kernel(
        _body,
        out_type=jax.ShapeDtypeStruct((M, DP), jnp.float32),
        mesh=mesh,
        compiler_params=pltpu.CompilerParams(
            needs_layout_passes=False, use_tc_tiling_on_sc=False
        ),
        scratch_types=[
            pltpu.VMEM((SLICE,), jnp.int32),
            pltpu.VMEM((PV,), jnp.int32),
            pltpu.VMEM((NH,), jnp.int32),
            pltpu.VMEM((NH,), jnp.int32),
            pltpu.VMEM((1, C), jnp.int32),
            pltpu.VMEM((C, DP), jnp.float32),
            pltpu.VMEM_SHARED((R + 1, DP), jnp.float32),
            pltpu.SemaphoreType.DMA,
        ],
    )(mem64, val64, idx)
    return out64[:, :D]


def kernel(mem, val, idx):
    return _scatter_add(mem, val, idx)


# speedup vs baseline: 1.2015x; 1.2015x over previous
import jax
import jax.numpy as jnp
from jax import lax
from jax.experimental import pallas as pl
from jax.experimental.pallas import tpu as pltpu
from jax.experimental.pallas import tpu_sc as plsc

M = 1000000
D = 59
DP = 64
B = 262144

NC = 2
NS = 16
L = 16

R = 16384
RSH = 14
NFULL = M // R
REM = M - NFULL * R
NB = 62
SPLIT = 31

SLICE = B // NS
NV = SLICE // L
C = 128
TRASH = R
ALIGN_SLACK = SPLIT * 8
PV = SLICE + ALIGN_SLACK + C + L
TRASHB = PV - L

ROWS_PER_TILE = R // NS
NH = NB * L


def _body(mem, val, idx, out, idx_v, pos_v, hist_v, cnt_v, lidx, rows, acc, gsem):
    c = lax.axis_index("c")
    s = lax.axis_index("s")
    tile_base = s * SLICE
    blk_lo = SPLIT * c

    pltpu.sync_copy(idx.at[pl.ds(tile_base, SLICE)], idx_v)

    def _prefill(i, _):
        pos_v[pl.ds(i * L, L)] = jnp.zeros((L,), jnp.int32) + tile_base
        return 0

    lax.fori_loop(0, PV // L, _prefill, 0)

    def _zero(i, _):
        hist_v[pl.ds(i * L, L)] = jnp.zeros((L,), jnp.int32)
        return 0

    lax.fori_loop(0, NH // L, _zero, 0)

    def _hist(i, _):
        lanes = lax.iota(jnp.int32, L)
        iv = idx_v[pl.ds(i * L, L)]
        blk = lax.shift_right_logical(iv, RSH)
        cidx = blk * L + lanes
        plsc.addupdate_scatter(hist_v, [cidx], jnp.zeros((L,), jnp.int32) + 1)
        return 0

    lax.fori_loop(0, NV, _hist, 0)

    def _bases(j, base):
        lanes = lax.iota(jnp.int32, L)
        hv = hist_v[pl.ds(j * L, L)]
        o = j - blk_lo
        own = (o >= 0) & (o < SPLIT)
        csum = plsc.cumsum(hv)
        excl = csum - hv
        cur = jnp.where(own, base + excl, TRASHB + lanes)
        cnt_v[pl.ds(j * L, L)] = cur
        cnt8 = (jnp.sum(hv) + 7) & jnp.int32(-8)
        return base + jnp.where(own, cnt8, 0)

    lax.fori_loop(0, NB, _bases, jnp.int32(0))

    def _rank(i, _):
        lanes = lax.iota(jnp.int32, L)
        iv = idx_v[pl.ds(i * L, L)]
        blk = lax.shift_right_logical(iv, RSH)
        o = blk - blk_lo
        own = ((o >= 0) & (o < SPLIT)).astype(jnp.int32)
        cidx = blk * L + lanes
        cur = plsc.load_gather(cnt_v, [cidx])
        posvec = (tile_base + i * L) + lanes
        plsc.store_scatter(pos_v, [cur], posvec)
        plsc.addupdate_scatter(cnt_v, [cidx], own)
        return 0

    lax.fori_loop(0, NV, _rank, 0)

    def _block(k, base):
        blk = blk_lo + k
        lo = blk * R
        isp = blk >= NFULL

        @pl.when(jnp.logical_not(isp))
        def _():
            pltpu.sync_copy(
                mem.at[pl.ds(lo + s * ROWS_PER_TILE, ROWS_PER_TILE)],
                acc.at[pl.ds(s * ROWS_PER_TILE, ROWS_PER_TILE)],
            )

        @pl.when(isp & (s == 0))
        def _():
            pltpu.sync_copy(mem.at[pl.ds(lo, REM)], acc.at[pl.ds(0, REM)])

        plsc.subcore_barrier()

        cnt = jnp.sum(hist_v[pl.ds(blk * L, L)])
        end = base + cnt

        def _chunk(j, _):
            c0 = pl.multiple_of(base + j * C, 8)
            for kk in range(C // L):
                lanes = lax.iota(jnp.int32, L)
                slot = c0 + kk * L + lanes
                pv = pos_v[pl.ds(c0 + kk * L, L)]
                ival = plsc.load_gather(idx_v, [pv - tile_base])
                li = ival - lo
                li = jnp.where(slot < end, li, TRASH)
                lidx[0, pl.ds(kk * L, L)] = li
            pltpu.async_copy(val.at[pos_v.at[pl.ds(c0, C)]], rows, gsem).wait()
            pltpu.sync_copy(rows, acc.at[lidx.at[0]], add=True)
            return 0

        nch = lax.div(cnt + (C - 1), jnp.int32(C))
        lax.fori_loop(0, nch, _chunk, 0)

        plsc.subcore_barrier()

        @pl.when(jnp.logical_not(isp))
        def _():
            pltpu.sync_copy(
                acc.at[pl.ds(s * ROWS_PER_TILE, ROWS_PER_TILE)],
                out.at[pl.ds(lo + s * ROWS_PER_TILE, ROWS_PER_TILE)],
            )

        @pl.when(isp & (s == 0))
        def _():
            pltpu.sync_copy(acc.at[pl.ds(0, REM)], out.at[pl.ds(lo, REM)])

        return base + ((cnt + 7) & jnp.int32(-8))

    lax.fori_loop(0, SPLIT, _block, jnp.int32(0))


@jax.jit
def _scatter_add(mem, val, idx):
    mem64 = jnp.pad(mem, ((0, 0), (0, DP - D)))
    val64 = jnp.pad(val, ((0, 0), (0, DP - D)))
    mesh = plsc.VectorSubcoreMesh(core_axis_name="c", subcore_axis_name="s")
    out64 = pl.kernel(
        _body,
        out_type=jax.ShapeDtypeStruct((M, DP), jnp.float32),
        mesh=mesh,
        compiler_params=pltpu.CompilerParams(
            needs_layout_passes=False, use_tc_tiling_on_sc=False
        ),
        scratch_types=[
            pltpu.VMEM((SLICE,), jnp.int32),
            pltpu.VMEM((PV,), jnp.int32),
            pltpu.VMEM((NH,), jnp.int32),
            pltpu.VMEM((NH,), jnp.int32),
            pltpu.VMEM((1, C), jnp.int32),
            pltpu.VMEM((C, DP), jnp.float32),
            pltpu.VMEM_SHARED((R + 1, DP), jnp.float32),
            pltpu.SemaphoreType.DMA,
        ],
    )(mem64, val64, idx)
    return out64[:, :D]


def kernel(mem, val, idx):
    return _scatter_add(mem, val, idx)
